# baseline (speedup 1.0000x reference)
import jax
import jax.numpy as jnp
from jax.experimental import pallas as pl
from jax.experimental.pallas import tpu as pltpu

_ENT_BLOCK = 2000


def _copy_block(src_ref, dst_ref):
    dst_ref[...] = src_ref[...]


def kernel(emb_ent, emb_rel, edge_index, rel, edge_index_all, rel_all):
    n, d = emb_ent.shape
    ent_out = pl.pallas_call(
        _copy_block,
        grid=(n // _ENT_BLOCK,),
        in_specs=[pl.BlockSpec((_ENT_BLOCK, d), lambda i: (i, 0))],
        out_specs=pl.BlockSpec((_ENT_BLOCK, d), lambda i: (i, 0)),
        out_shape=jax.ShapeDtypeStruct((n, d), emb_ent.dtype),
        compiler_params=pltpu.CompilerParams(
            dimension_semantics=("parallel",)),
    )(emb_ent)
    rel_out = pl.pallas_call(
        _copy_block,
        out_shape=jax.ShapeDtypeStruct(emb_rel.shape, emb_rel.dtype),
    )(emb_rel)
    return (ent_out, rel_out)
